# baseline (speedup 1.0000x reference)
import functools

import jax
import jax.numpy as jnp
from jax import lax
from jax.experimental import pallas as pl
from jax.experimental.pallas import tpu as pltpu
from jax.experimental.pallas import tpu_sc as plsc

BATCH = 4096
FIELDS = 26
VOCAB = 100000
DIM = 32

NUM_CORES = 2
NUM_SUBCORES = 16
LANES = 16
NW = NUM_CORES * NUM_SUBCORES
UNITS = FIELDS * DIM
UNITS_PER_W = UNITS // NW

VMAIN = (VOCAB // 128) * 128
HALF = 50048
BOUNDS0 = [0, 6272, 12544, 18816, 25088, 31360, 37632, 43904, HALF]
BOUNDS1 = [HALF, 56320, 62592, 68864, 75136, 81408, 87680, 93952, VMAIN]
B1LEN = VMAIN - HALF
HBUF = B1LEN + 128

_mesh = plsc.VectorSubcoreMesh(core_axis_name="c", subcore_axis_name="s")


@functools.partial(
    pl.kernel,
    mesh=_mesh,
    out_type=jax.ShapeDtypeStruct((UNITS, BATCH), jnp.float32),
    scratch_types=[
        pltpu.VMEM((HALF,), jnp.float32),
        pltpu.VMEM((HBUF,), jnp.float32),
        pltpu.VMEM((BATCH,), jnp.int32),
        pltpu.VMEM((2, BATCH), jnp.float32),
        pltpu.SemaphoreType.DMA,
        pltpu.SemaphoreType.DMA,
        pltpu.SemaphoreType.DMA,
    ],
    compiler_params=pltpu.CompilerParams(
        use_tc_tiling_on_sc=True, needs_layout_passes=False
    ),
)
def _gather_kernel(
    idx_hbm, tab_hbm, tail_hbm, out_hbm, h0_v, h1_v, idx_v, o_v,
    sem0, sem1, osem,
):
    wid = lax.axis_index("s") * NUM_CORES + lax.axis_index("c")

    def half0_copies(u):
        f = u // DIM
        e = u % DIM
        return [
            pltpu.make_async_copy(
                tab_hbm.at[f, e, pl.ds(st, en - st)],
                h0_v.at[pl.ds(st, en - st)],
                sem0,
            )
            for st, en in zip(BOUNDS0[:-1], BOUNDS0[1:])
        ]

    def half1_copies(u):
        f = u // DIM
        e = u % DIM
        cps = [
            pltpu.make_async_copy(
                tab_hbm.at[f, e, pl.ds(st, en - st)],
                h1_v.at[pl.ds(st - HALF, en - st)],
                sem1,
            )
            for st, en in zip(BOUNDS1[:-1], BOUNDS1[1:])
        ]
        cps.append(
            pltpu.make_async_copy(
                tail_hbm.at[f, e], h1_v.at[pl.ds(B1LEN, 128)], sem1
            )
        )
        return cps

    u0 = wid * UNITS_PER_W
    for cp in half0_copies(u0):
        cp.start()
    for cp in half1_copies(u0):
        cp.start()

    @pl.loop(0, UNITS_PER_W)
    def _unit(k):
        u = u0 + k
        e = u % DIM
        b = k % 2

        @pl.when((k == 0) | (e == 0))
        def _():
            pltpu.sync_copy(idx_hbm.at[u // DIM], idx_v)

        @pl.when(k >= 2)
        def _():
            pltpu.make_async_copy(o_v.at[b], out_hbm.at[u], osem).wait()

        for cp in half0_copies(u):
            cp.wait()

        @pl.loop(0, BATCH // LANES, unroll=8)
        def _g0(g):
            sl = pl.ds(g * LANES, LANES)
            iv = idx_v[sl]
            o_v[b, sl] = plsc.load_gather(h0_v, [jnp.minimum(iv, HALF - 1)])

        @pl.when(k + 1 < UNITS_PER_W)
        def _():
            for cp in half0_copies(u + 1):
                cp.start()

        for cp in half1_copies(u):
            cp.wait()

        @pl.loop(0, BATCH // LANES, unroll=8)
        def _g1(g):
            sl = pl.ds(g * LANES, LANES)
            iv = idx_v[sl]
            hi = plsc.load_gather(
                h1_v, [jnp.maximum(iv - HALF, 0)]
            )
            o_v[b, sl] = jnp.where(iv >= HALF, hi, o_v[b, sl])

        @pl.when(k + 1 < UNITS_PER_W)
        def _():
            for cp in half1_copies(u + 1):
                cp.start()

        pltpu.make_async_copy(o_v.at[b], out_hbm.at[u], osem).start()

    for t in range(2):
        pltpu.make_async_copy(
            o_v.at[t], out_hbm.at[u0 + t], osem
        ).wait()


def kernel(indices, tables):
    idx_t = indices.astype(jnp.int32).T
    tab_t = jnp.transpose(tables, (0, 2, 1))
    tail = jnp.pad(tab_t[:, :, VMAIN:], ((0, 0), (0, 0), (0, 96)))
    out = _gather_kernel(idx_t, tab_t, tail)
    return out.T.reshape(BATCH, FIELDS * DIM)


# speedup vs baseline: 1.0061x; 1.0061x over previous
import functools

import jax
import jax.numpy as jnp
from jax import lax
from jax.experimental import pallas as pl
from jax.experimental.pallas import tpu as pltpu
from jax.experimental.pallas import tpu_sc as plsc

BATCH = 4096
FIELDS = 26
VOCAB = 100000
DIM = 32

NUM_CORES = 2
NUM_SUBCORES = 16
LANES = 16
NW = NUM_CORES * NUM_SUBCORES
UNITS = FIELDS * DIM
UNITS_PER_W = UNITS // NW

VMAIN = (VOCAB // 128) * 128
HALF = 50048
BOUNDS0 = [0, 12544, 25088, 37632, HALF]
BOUNDS1 = [HALF, 62592, 75136, 87680, VMAIN]
B1LEN = VMAIN - HALF
HBUF = B1LEN + 128

_mesh = plsc.VectorSubcoreMesh(core_axis_name="c", subcore_axis_name="s")


@functools.partial(
    pl.kernel,
    mesh=_mesh,
    out_type=jax.ShapeDtypeStruct((UNITS, BATCH), jnp.float32),
    scratch_types=[
        pltpu.VMEM((HALF,), jnp.float32),
        pltpu.VMEM((HBUF,), jnp.float32),
        pltpu.VMEM((BATCH,), jnp.int32),
        pltpu.VMEM((2, BATCH), jnp.float32),
        pltpu.SemaphoreType.DMA,
        pltpu.SemaphoreType.DMA,
        pltpu.SemaphoreType.DMA,
    ],
    compiler_params=pltpu.CompilerParams(
        use_tc_tiling_on_sc=True, needs_layout_passes=False
    ),
)
def _gather_kernel(
    idx_hbm, tab_hbm, tail_hbm, out_hbm, h0_v, h1_v, idx_v, o_v,
    sem0, sem1, osem,
):
    wid = lax.axis_index("s") * NUM_CORES + lax.axis_index("c")

    def half0_copies(u):
        f = u // DIM
        e = u % DIM
        return [
            pltpu.make_async_copy(
                tab_hbm.at[f, e, pl.ds(st, en - st)],
                h0_v.at[pl.ds(st, en - st)],
                sem0,
            )
            for st, en in zip(BOUNDS0[:-1], BOUNDS0[1:])
        ]

    def half1_copies(u):
        f = u // DIM
        e = u % DIM
        cps = [
            pltpu.make_async_copy(
                tab_hbm.at[f, e, pl.ds(st, en - st)],
                h1_v.at[pl.ds(st - HALF, en - st)],
                sem1,
            )
            for st, en in zip(BOUNDS1[:-1], BOUNDS1[1:])
        ]
        cps.append(
            pltpu.make_async_copy(
                tail_hbm.at[f, e], h1_v.at[pl.ds(B1LEN, 128)], sem1
            )
        )
        return cps

    u0 = wid * UNITS_PER_W
    for cp in half0_copies(u0):
        cp.start()
    for cp in half1_copies(u0):
        cp.start()

    @pl.loop(0, UNITS_PER_W)
    def _unit(k):
        u = u0 + k
        e = u % DIM
        b = k % 2

        @pl.when((k == 0) | (e == 0))
        def _():
            pltpu.sync_copy(idx_hbm.at[u // DIM], idx_v)

        @pl.when(k >= 2)
        def _():
            pltpu.make_async_copy(o_v.at[b], out_hbm.at[u], osem).wait()

        for cp in half0_copies(u):
            cp.wait()

        @pl.loop(0, BATCH // LANES, unroll=8)
        def _g0(g):
            sl = pl.ds(g * LANES, LANES)
            iv = idx_v[sl]
            o_v[b, sl] = plsc.load_gather(h0_v, [jnp.minimum(iv, HALF - 1)])

        @pl.when(k + 1 < UNITS_PER_W)
        def _():
            for cp in half0_copies(u + 1):
                cp.start()

        for cp in half1_copies(u):
            cp.wait()

        @pl.loop(0, BATCH // LANES, unroll=8)
        def _g1(g):
            sl = pl.ds(g * LANES, LANES)
            iv = idx_v[sl]
            hi = plsc.load_gather(
                h1_v, [jnp.maximum(iv - HALF, 0)]
            )
            o_v[b, sl] = jnp.where(iv >= HALF, hi, o_v[b, sl])

        @pl.when(k + 1 < UNITS_PER_W)
        def _():
            for cp in half1_copies(u + 1):
                cp.start()

        pltpu.make_async_copy(o_v.at[b], out_hbm.at[u], osem).start()

    for t in range(2):
        pltpu.make_async_copy(
            o_v.at[t], out_hbm.at[u0 + t], osem
        ).wait()


def kernel(indices, tables):
    idx_t = indices.astype(jnp.int32).T
    tab_t = jnp.transpose(tables, (0, 2, 1))
    tail = jnp.pad(tab_t[:, :, VMAIN:], ((0, 0), (0, 0), (0, 96)))
    out = _gather_kernel(idx_t, tab_t, tail)
    return out.T.reshape(BATCH, FIELDS * DIM)
